# baseline (speedup 1.0000x reference)
import jax
import jax.numpy as jnp
from jax import lax
from jax.experimental import pallas as pl
from jax.experimental.pallas import tpu as pltpu
from jax.experimental.pallas import tpu_sc as plsc

N_VERT = 10000
N_ELEM = 160000
NC, NS, L = 2, 16, 16
NW = NC * NS
PER_W = N_ELEM // NW
CHUNK = 1000
SLOT = 1024
N_CHUNKS_W = PER_W // CHUNK
N_CHUNKS = N_ELEM // CHUNK
STEPS = (CHUNK + L - 1) // L
NB = 12
ESTRIDE = 4 * SLOT
FSTRIDE = NB * SLOT

_LN2 = 0.6931471805599453
_SQRT2 = 1.4142135623730951


def _vlog(x):
    bits = lax.bitcast_convert_type(x, jnp.int32)
    e = jnp.right_shift(bits, 23) - 127
    m_bits = jnp.bitwise_or(jnp.bitwise_and(bits, 0x007FFFFF), 0x3F800000)
    m = lax.bitcast_convert_type(m_bits, jnp.float32)
    big = m > _SQRT2
    m = jnp.where(big, m * 0.5, m)
    e = jnp.where(big, e + 1, e)
    t = (m - 1.0) / (m + 1.0)
    t2 = t * t
    p = 2.0 * t * (1.0 + t2 * (1.0 / 3.0 + t2 * (1.0 / 5.0 + t2 * (1.0 / 7.0))))
    return e.astype(jnp.float32) * _LN2 + p


U = 3
ITERS = -(-CHUNK // (L * U))


def _sc_body(et_hbm, bt_hbm, px_hbm, py_hbm, pz_hbm, A_hbm, B_hbm, al_hbm,
             out_hbm, px_v, py_v, pz_v, ebuf_v, fbuf_v, A_v, B_v, al_v,
             acc_v, sem0, sem1, psem):
    c = lax.axis_index("c")
    s = lax.axis_index("s")
    wid = s * NC + c
    sems = (sem0, sem1)

    pltpu.async_copy(px_hbm, px_v, psem)
    pltpu.async_copy(py_hbm, py_v, psem)
    pltpu.async_copy(pz_hbm, pz_v, psem)
    lanes = lax.broadcasted_iota(jnp.int32, (L,), 0)

    def issue(ci, slot):
        g = wid * N_CHUNKS_W + ci
        base = g * CHUNK
        sem = sems[slot]
        return [
            pltpu.async_copy(et_hbm.at[pl.ds(0, 4), pl.ds(base, CHUNK)],
                             ebuf_v.at[slot, pl.ds(0, 4), pl.ds(0, CHUNK)], sem),
            pltpu.async_copy(bt_hbm.at[pl.ds(0, NB), pl.ds(base, CHUNK)],
                             fbuf_v.at[slot, pl.ds(0, NB), pl.ds(0, CHUNK)], sem),
            pltpu.async_copy(A_hbm.at[pl.ds(base, CHUNK)],
                             A_v.at[slot, pl.ds(0, CHUNK)], sem),
            pltpu.async_copy(B_hbm.at[pl.ds(base, CHUNK)],
                             B_v.at[slot, pl.ds(0, CHUNK)], sem),
            pltpu.async_copy(al_hbm.at[pl.ds(base, CHUNK)],
                             al_v.at[slot, pl.ds(0, CHUNK)], sem),
        ]

    descs = issue(0, 0)
    pltpu.make_async_copy(px_hbm, px_v, psem).wait()
    pltpu.make_async_copy(py_hbm, py_v, psem).wait()
    pltpu.make_async_copy(pz_hbm, pz_v, psem).wait()
    acc = jnp.zeros((L,), jnp.float32)

    for ci in range(N_CHUNKS_W):
        slot = ci % 2
        for d in descs:
            d.wait()
        if ci + 1 < N_CHUNKS_W:
            descs = issue(ci + 1, 1 - slot)

        def step(si, acc, slot=slot):
            for u in range(U):
                off = si * (L * U) + u * L
                valid = off + lanes < CHUNK
                ev = [jnp.minimum(jnp.maximum(
                          ebuf_v[slot, f, pl.ds(off, L)], 0), N_VERT - 1)
                      for f in range(4)]
                p = [[plsc.load_gather(tb, [ev[f]])
                      for tb in (px_v, py_v, pz_v)] for f in range(4)]
                b = [[fbuf_v[slot, 3 * f + t, pl.ds(off, L)]
                      for t in range(3)] for f in range(4)]
                A = A_v[slot, pl.ds(off, L)]
                B = B_v[slot, pl.ds(off, L)]
                alpha = al_v[slot, pl.ds(off, L)]
                F = [[p[0][t] * b[0][l] + p[1][t] * b[1][l]
                      + p[2][t] * b[2][l] + p[3][t] * b[3][l]
                      for l in range(3)] for t in range(3)]
                Ic = (F[0][0] * F[0][0] + F[0][1] * F[0][1] + F[0][2] * F[0][2]
                      + F[1][0] * F[1][0] + F[1][1] * F[1][1] + F[1][2] * F[1][2]
                      + F[2][0] * F[2][0] + F[2][1] * F[2][1] + F[2][2] * F[2][2])
                J = (F[0][0] * (F[1][1] * F[2][2] - F[1][2] * F[2][1])
                     - F[0][1] * (F[1][0] * F[2][2] - F[1][2] * F[2][0])
                     + F[0][2] * (F[1][0] * F[2][1] - F[1][1] * F[2][0]))
                ic_v = jnp.maximum(Ic + 1.0, 0.0) + 1e-30
                d = J - alpha
                psi_w = (A * ((Ic - 3.0) - _vlog(ic_v)) + B * (d * d))
                acc = acc + jnp.where(valid, psi_w, 0.0)
            return acc

        acc = lax.fori_loop(0, ITERS, step, acc)

    acc_v[...] = acc
    pltpu.sync_copy(acc_v, out_hbm.at[pl.ds(wid * L, L)])


@jax.jit
def kernel(position, elements, polynomials, lam, mu, measure):
    et = elements.T
    bt = polynomials[:, :, :3].reshape(N_ELEM, NB).T
    px, py, pz = position[:, 0], position[:, 1], position[:, 2]
    A = 0.5 * mu * measure
    B = 0.5 * lam * measure
    alpha = 0.75 * mu / lam + 1.0

    mesh = plsc.VectorSubcoreMesh(core_axis_name="c", subcore_axis_name="s",
                                  num_cores=NC, num_subcores=NS)
    partials = pl.kernel(
        _sc_body,
        out_type=jax.ShapeDtypeStruct((NW * L,), jnp.float32),
        mesh=mesh,
        compiler_params=pltpu.CompilerParams(needs_layout_passes=False,
                                             use_tc_tiling_on_sc=False),
        scratch_types=[
            pltpu.VMEM((N_VERT,), jnp.float32),
            pltpu.VMEM((N_VERT,), jnp.float32),
            pltpu.VMEM((N_VERT,), jnp.float32),
            pltpu.VMEM((2, 4, SLOT), jnp.int32),
            pltpu.VMEM((2, NB, SLOT), jnp.float32),
            pltpu.VMEM((2, SLOT), jnp.float32),
            pltpu.VMEM((2, SLOT), jnp.float32),
            pltpu.VMEM((2, SLOT), jnp.float32),
            pltpu.VMEM((L,), jnp.float32),
            pltpu.SemaphoreType.DMA,
            pltpu.SemaphoreType.DMA,
            pltpu.SemaphoreType.DMA,
        ],
    )(et, bt, px, py, pz, A, B, alpha)
    return jnp.sum(partials)


# speedup vs baseline: 1.0113x; 1.0113x over previous
import jax
import jax.numpy as jnp
from jax import lax
from jax.experimental import pallas as pl
from jax.experimental.pallas import tpu as pltpu
from jax.experimental.pallas import tpu_sc as plsc

N_VERT = 10000
N_ELEM = 160000
NC, NS, L = 2, 16, 16
NW = NC * NS
PER_W = N_ELEM // NW
CHUNK = 1000
SLOT = 1024
N_CHUNKS_W = PER_W // CHUNK
N_CHUNKS = N_ELEM // CHUNK
STEPS = (CHUNK + L - 1) // L
NB = 12
ESTRIDE = 4 * SLOT
FSTRIDE = NB * SLOT

_LN2 = 0.6931471805599453
_SQRT2 = 1.4142135623730951


def _vlog(x):
    bits = lax.bitcast_convert_type(x, jnp.int32)
    e = jnp.right_shift(bits, 23) - 127
    m_bits = jnp.bitwise_or(jnp.bitwise_and(bits, 0x007FFFFF), 0x3F800000)
    m = lax.bitcast_convert_type(m_bits, jnp.float32)
    big = m > _SQRT2
    m = jnp.where(big, m * 0.5, m)
    e = jnp.where(big, e + 1, e)
    t = (m - 1.0) / (m + 1.0)
    t2 = t * t
    p = 2.0 * t * (1.0 + t2 * (1.0 / 3.0 + t2 * (1.0 / 5.0 + t2 * (1.0 / 7.0))))
    return e.astype(jnp.float32) * _LN2 + p


U = 2
ITERS = -(-CHUNK // (L * U))


def _sc_body(et_hbm, bt_hbm, px_hbm, py_hbm, pz_hbm, A_hbm, B_hbm, al_hbm,
             out_hbm, px_v, py_v, pz_v, ebuf_v, fbuf_v, A_v, B_v, al_v,
             acc_v, sem0, sem1, psem):
    c = lax.axis_index("c")
    s = lax.axis_index("s")
    wid = s * NC + c
    sems = (sem0, sem1)

    pltpu.async_copy(px_hbm, px_v, psem)
    pltpu.async_copy(py_hbm, py_v, psem)
    pltpu.async_copy(pz_hbm, pz_v, psem)
    lanes = lax.broadcasted_iota(jnp.int32, (L,), 0)

    def issue(ci, slot):
        g = wid * N_CHUNKS_W + ci
        base = g * CHUNK
        sem = sems[slot]
        return [
            pltpu.async_copy(et_hbm.at[pl.ds(0, 4), pl.ds(base, CHUNK)],
                             ebuf_v.at[slot, pl.ds(0, 4), pl.ds(0, CHUNK)], sem),
            pltpu.async_copy(bt_hbm.at[pl.ds(0, NB), pl.ds(base, CHUNK)],
                             fbuf_v.at[slot, pl.ds(0, NB), pl.ds(0, CHUNK)], sem),
            pltpu.async_copy(A_hbm.at[pl.ds(base, CHUNK)],
                             A_v.at[slot, pl.ds(0, CHUNK)], sem),
            pltpu.async_copy(B_hbm.at[pl.ds(base, CHUNK)],
                             B_v.at[slot, pl.ds(0, CHUNK)], sem),
            pltpu.async_copy(al_hbm.at[pl.ds(base, CHUNK)],
                             al_v.at[slot, pl.ds(0, CHUNK)], sem),
        ]

    descs = issue(0, 0)
    pltpu.make_async_copy(px_hbm, px_v, psem).wait()
    pltpu.make_async_copy(py_hbm, py_v, psem).wait()
    pltpu.make_async_copy(pz_hbm, pz_v, psem).wait()
    acc = jnp.zeros((L,), jnp.float32)

    for ci in range(N_CHUNKS_W):
        slot = ci % 2
        for d in descs:
            d.wait()
        if ci + 1 < N_CHUNKS_W:
            descs = issue(ci + 1, 1 - slot)

        def step(si, acc, slot=slot):
            for u in range(U):
                off = si * (L * U) + u * L
                valid = off + lanes < CHUNK
                ev = [jnp.minimum(jnp.maximum(
                          ebuf_v[slot, f, pl.ds(off, L)], 0), N_VERT - 1)
                      for f in range(4)]
                p = [[plsc.load_gather(tb, [ev[f]])
                      for tb in (px_v, py_v, pz_v)] for f in range(4)]
                b = [[fbuf_v[slot, 3 * f + t, pl.ds(off, L)]
                      for t in range(3)] for f in range(4)]
                A = A_v[slot, pl.ds(off, L)]
                B = B_v[slot, pl.ds(off, L)]
                alpha = al_v[slot, pl.ds(off, L)]
                F = [[p[0][t] * b[0][l] + p[1][t] * b[1][l]
                      + p[2][t] * b[2][l] + p[3][t] * b[3][l]
                      for l in range(3)] for t in range(3)]
                Ic = (F[0][0] * F[0][0] + F[0][1] * F[0][1] + F[0][2] * F[0][2]
                      + F[1][0] * F[1][0] + F[1][1] * F[1][1] + F[1][2] * F[1][2]
                      + F[2][0] * F[2][0] + F[2][1] * F[2][1] + F[2][2] * F[2][2])
                J = (F[0][0] * (F[1][1] * F[2][2] - F[1][2] * F[2][1])
                     - F[0][1] * (F[1][0] * F[2][2] - F[1][2] * F[2][0])
                     + F[0][2] * (F[1][0] * F[2][1] - F[1][1] * F[2][0]))
                ic_v = jnp.maximum(Ic + 1.0, 0.0) + 1e-30
                d = J - alpha
                psi_w = (A * ((Ic - 3.0) - _vlog(ic_v)) + B * (d * d))
                acc = acc + jnp.where(valid, psi_w, 0.0)
            return acc

        acc = lax.fori_loop(0, ITERS, step, acc)

    acc_v[...] = acc
    pltpu.sync_copy(acc_v, out_hbm.at[pl.ds(wid * L, L)])


@jax.jit
def kernel(position, elements, polynomials, lam, mu, measure):
    et = elements.T
    bt = polynomials[:, :, :3].reshape(N_ELEM, NB).T
    px, py, pz = position[:, 0], position[:, 1], position[:, 2]
    A = 0.5 * mu * measure
    B = 0.5 * lam * measure
    alpha = 0.75 * mu / lam + 1.0

    mesh = plsc.VectorSubcoreMesh(core_axis_name="c", subcore_axis_name="s",
                                  num_cores=NC, num_subcores=NS)
    partials = pl.kernel(
        _sc_body,
        out_type=jax.ShapeDtypeStruct((NW * L,), jnp.float32),
        mesh=mesh,
        compiler_params=pltpu.CompilerParams(needs_layout_passes=False,
                                             use_tc_tiling_on_sc=False),
        scratch_types=[
            pltpu.VMEM((N_VERT,), jnp.float32),
            pltpu.VMEM((N_VERT,), jnp.float32),
            pltpu.VMEM((N_VERT,), jnp.float32),
            pltpu.VMEM((2, 4, SLOT), jnp.int32),
            pltpu.VMEM((2, NB, SLOT), jnp.float32),
            pltpu.VMEM((2, SLOT), jnp.float32),
            pltpu.VMEM((2, SLOT), jnp.float32),
            pltpu.VMEM((2, SLOT), jnp.float32),
            pltpu.VMEM((L,), jnp.float32),
            pltpu.SemaphoreType.DMA,
            pltpu.SemaphoreType.DMA,
            pltpu.SemaphoreType.DMA,
        ],
    )(et, bt, px, py, pz, A, B, alpha)
    return jnp.sum(partials)


# speedup vs baseline: 1.0528x; 1.0411x over previous
import jax
import jax.numpy as jnp
from jax import lax
from jax.experimental import pallas as pl
from jax.experimental.pallas import tpu as pltpu
from jax.experimental.pallas import tpu_sc as plsc

N_VERT = 10000
N_ELEM = 160000
NC, NS, L = 2, 16, 16
NW = NC * NS
PER_W = N_ELEM // NW
CHUNK = 1000
SLOT = 1024
N_CHUNKS_W = PER_W // CHUNK
N_CHUNKS = N_ELEM // CHUNK
STEPS = (CHUNK + L - 1) // L
NB = 12
ESTRIDE = 4 * SLOT
FSTRIDE = NB * SLOT

_LN2 = 0.6931471805599453
_SQRT2 = 1.4142135623730951


def _vlog(x):
    bits = lax.bitcast_convert_type(x, jnp.int32)
    e = jnp.right_shift(bits, 23) - 127
    m_bits = jnp.bitwise_or(jnp.bitwise_and(bits, 0x007FFFFF), 0x3F800000)
    m = lax.bitcast_convert_type(m_bits, jnp.float32)
    big = m > _SQRT2
    m = jnp.where(big, m * 0.5, m)
    e = jnp.where(big, e + 1, e)
    t = (m - 1.0) / (m + 1.0)
    t2 = t * t
    p = 2.0 * t * (1.0 + t2 * (1.0 / 3.0 + t2 * (1.0 / 5.0 + t2 * (1.0 / 7.0))))
    return e.astype(jnp.float32) * _LN2 + p


U = 2
ITERS = -(-CHUNK // (L * U))


def _sc_body(et_hbm, bt_hbm, px_hbm, py_hbm, pz_hbm, lam_hbm, mu_hbm, meas_hbm,
             out_hbm, px_v, py_v, pz_v, ebuf_v, fbuf_v, lam_v, mu_v, meas_v,
             acc_v, sem0, sem1, psem):
    c = lax.axis_index("c")
    s = lax.axis_index("s")
    wid = s * NC + c
    sems = (sem0, sem1)

    pltpu.async_copy(px_hbm, px_v, psem)
    pltpu.async_copy(py_hbm, py_v, psem)
    pltpu.async_copy(pz_hbm, pz_v, psem)
    lanes = lax.broadcasted_iota(jnp.int32, (L,), 0)

    def issue(ci, slot):
        g = wid * N_CHUNKS_W + ci
        base = g * CHUNK
        sem = sems[slot]
        return [
            pltpu.async_copy(et_hbm.at[pl.ds(0, 4), pl.ds(base, CHUNK)],
                             ebuf_v.at[slot, pl.ds(0, 4), pl.ds(0, CHUNK)], sem),
            pltpu.async_copy(bt_hbm.at[pl.ds(0, NB), pl.ds(base, CHUNK)],
                             fbuf_v.at[slot, pl.ds(0, NB), pl.ds(0, CHUNK)], sem),
            pltpu.async_copy(lam_hbm.at[pl.ds(base, CHUNK)],
                             lam_v.at[slot, pl.ds(0, CHUNK)], sem),
            pltpu.async_copy(mu_hbm.at[pl.ds(base, CHUNK)],
                             mu_v.at[slot, pl.ds(0, CHUNK)], sem),
            pltpu.async_copy(meas_hbm.at[pl.ds(base, CHUNK)],
                             meas_v.at[slot, pl.ds(0, CHUNK)], sem),
        ]

    descs = issue(0, 0)
    pltpu.make_async_copy(px_hbm, px_v, psem).wait()
    pltpu.make_async_copy(py_hbm, py_v, psem).wait()
    pltpu.make_async_copy(pz_hbm, pz_v, psem).wait()
    acc = jnp.zeros((L,), jnp.float32)

    for ci in range(N_CHUNKS_W):
        slot = ci % 2
        for d in descs:
            d.wait()
        if ci + 1 < N_CHUNKS_W:
            descs = issue(ci + 1, 1 - slot)

        def step(si, acc, slot=slot):
            for u in range(U):
                off = si * (L * U) + u * L
                valid = off + lanes < CHUNK
                ev = [jnp.minimum(jnp.maximum(
                          ebuf_v[slot, f, pl.ds(off, L)], 0), N_VERT - 1)
                      for f in range(4)]
                p = [[plsc.load_gather(tb, [ev[f]])
                      for tb in (px_v, py_v, pz_v)] for f in range(4)]
                b = [[fbuf_v[slot, 3 * f + t, pl.ds(off, L)]
                      for t in range(3)] for f in range(4)]
                lam = lam_v[slot, pl.ds(off, L)]
                mu = mu_v[slot, pl.ds(off, L)]
                meas = meas_v[slot, pl.ds(off, L)]
                F = [[p[0][t] * b[0][l] + p[1][t] * b[1][l]
                      + p[2][t] * b[2][l] + p[3][t] * b[3][l]
                      for l in range(3)] for t in range(3)]
                Ic = (F[0][0] * F[0][0] + F[0][1] * F[0][1] + F[0][2] * F[0][2]
                      + F[1][0] * F[1][0] + F[1][1] * F[1][1] + F[1][2] * F[1][2]
                      + F[2][0] * F[2][0] + F[2][1] * F[2][1] + F[2][2] * F[2][2])
                J = (F[0][0] * (F[1][1] * F[2][2] - F[1][2] * F[2][1])
                     - F[0][1] * (F[1][0] * F[2][2] - F[1][2] * F[2][0])
                     + F[0][2] * (F[1][0] * F[2][1] - F[1][1] * F[2][0]))
                ic_v = jnp.maximum(Ic + 1.0, 0.0) + 1e-30
                alpha = 0.75 * mu / lam + 1.0
                d = J - alpha
                psi = (0.5 * mu * (Ic - 3.0) + 0.5 * lam * (d * d)
                       - 0.5 * mu * _vlog(ic_v))
                acc = acc + jnp.where(valid, psi * meas, 0.0)
            return acc

        acc = lax.fori_loop(0, ITERS, step, acc)

    acc_v[...] = acc
    pltpu.sync_copy(acc_v, out_hbm.at[pl.ds(wid * L, L)])


@jax.jit
def kernel(position, elements, polynomials, lam, mu, measure):
    et = elements.T
    bt = polynomials[:, :, :3].reshape(N_ELEM, NB).T
    px, py, pz = position[:, 0], position[:, 1], position[:, 2]

    mesh = plsc.VectorSubcoreMesh(core_axis_name="c", subcore_axis_name="s",
                                  num_cores=NC, num_subcores=NS)
    partials = pl.kernel(
        _sc_body,
        out_type=jax.ShapeDtypeStruct((NW * L,), jnp.float32),
        mesh=mesh,
        compiler_params=pltpu.CompilerParams(needs_layout_passes=False,
                                             use_tc_tiling_on_sc=False),
        scratch_types=[
            pltpu.VMEM((N_VERT,), jnp.float32),
            pltpu.VMEM((N_VERT,), jnp.float32),
            pltpu.VMEM((N_VERT,), jnp.float32),
            pltpu.VMEM((2, 4, SLOT), jnp.int32),
            pltpu.VMEM((2, NB, SLOT), jnp.float32),
            pltpu.VMEM((2, SLOT), jnp.float32),
            pltpu.VMEM((2, SLOT), jnp.float32),
            pltpu.VMEM((2, SLOT), jnp.float32),
            pltpu.VMEM((L,), jnp.float32),
            pltpu.SemaphoreType.DMA,
            pltpu.SemaphoreType.DMA,
            pltpu.SemaphoreType.DMA,
        ],
    )(et, bt, px, py, pz, lam, mu, measure)
    return jnp.sum(partials)


# speedup vs baseline: 1.0548x; 1.0018x over previous
import jax
import jax.numpy as jnp
from jax import lax
from jax.experimental import pallas as pl
from jax.experimental.pallas import tpu as pltpu
from jax.experimental.pallas import tpu_sc as plsc

N_VERT = 10000
N_ELEM = 160000
NC, NS, L = 2, 16, 16
NW = NC * NS
PER_W = N_ELEM // NW
CHUNK = 1000
SLOT = 1024
N_CHUNKS_W = PER_W // CHUNK
NB = 12

_LN2 = 0.6931471805599453
_SQRT2 = 1.4142135623730951


def _vlog(x):
    bits = lax.bitcast_convert_type(x, jnp.int32)
    e = jnp.right_shift(bits, 23) - 127
    m_bits = jnp.bitwise_or(jnp.bitwise_and(bits, 0x007FFFFF), 0x3F800000)
    m = lax.bitcast_convert_type(m_bits, jnp.float32)
    big = m > _SQRT2
    m = jnp.where(big, m * 0.5, m)
    e = jnp.where(big, e + 1, e)
    t = (m - 1.0) / (m + 1.0)
    t2 = t * t
    p = 2.0 * t * (1.0 + t2 * (1.0 / 3.0 + t2 * (1.0 / 5.0 + t2 * (1.0 / 7.0))))
    return e.astype(jnp.float32) * _LN2 + p


U = 2
ITERS = -(-CHUNK // (L * U))


def _sc_body(et_hbm, bt_hbm, px_hbm, py_hbm, pz_hbm, lam_hbm, mu_hbm, meas_hbm,
             out_hbm, px_v, py_v, pz_v, ebuf_v, fbuf_v, lam_v, mu_v, meas_v,
             acc_v, sem0, sem1, psem):
    c = lax.axis_index("c")
    s = lax.axis_index("s")
    wid = s * NC + c
    sems = (sem0, sem1)

    pltpu.async_copy(px_hbm, px_v, psem)
    pltpu.async_copy(py_hbm, py_v, psem)
    pltpu.async_copy(pz_hbm, pz_v, psem)
    lanes = lax.broadcasted_iota(jnp.int32, (L,), 0)

    def issue(ci, slot):
        g = wid * N_CHUNKS_W + ci
        base = g * CHUNK
        sem = sems[slot]
        return [
            pltpu.async_copy(et_hbm.at[pl.ds(0, 4), pl.ds(base, CHUNK)],
                             ebuf_v.at[slot, pl.ds(0, 4), pl.ds(0, CHUNK)], sem),
            pltpu.async_copy(bt_hbm.at[pl.ds(0, NB), pl.ds(base, CHUNK)],
                             fbuf_v.at[slot, pl.ds(0, NB), pl.ds(0, CHUNK)], sem),
            pltpu.async_copy(lam_hbm.at[pl.ds(base, CHUNK)],
                             lam_v.at[slot, pl.ds(0, CHUNK)], sem),
            pltpu.async_copy(mu_hbm.at[pl.ds(base, CHUNK)],
                             mu_v.at[slot, pl.ds(0, CHUNK)], sem),
            pltpu.async_copy(meas_hbm.at[pl.ds(base, CHUNK)],
                             meas_v.at[slot, pl.ds(0, CHUNK)], sem),
        ]

    descs = issue(0, 0)
    pltpu.make_async_copy(px_hbm, px_v, psem).wait()
    pltpu.make_async_copy(py_hbm, py_v, psem).wait()
    pltpu.make_async_copy(pz_hbm, pz_v, psem).wait()
    acc = jnp.zeros((L,), jnp.float32)

    for ci in range(N_CHUNKS_W):
        slot = ci % 2
        for d in descs:
            d.wait()
        if ci + 1 < N_CHUNKS_W:
            descs = issue(ci + 1, 1 - slot)

        def step(si, acc, slot=slot):
            for u in range(U):
                off = si * (L * U) + u * L
                valid = off + lanes < CHUNK
                ev = [jnp.minimum(jnp.maximum(
                          ebuf_v[slot, f, pl.ds(off, L)], 0), N_VERT - 1)
                      for f in range(4)]
                p = [[plsc.load_gather(tb, [ev[f]])
                      for tb in (px_v, py_v, pz_v)] for f in range(4)]
                b = [[fbuf_v[slot, 3 * f + t, pl.ds(off, L)]
                      for t in range(3)] for f in range(4)]
                lam = lam_v[slot, pl.ds(off, L)]
                mu = mu_v[slot, pl.ds(off, L)]
                meas = meas_v[slot, pl.ds(off, L)]
                F = [[p[0][t] * b[0][l] + p[1][t] * b[1][l]
                      + p[2][t] * b[2][l] + p[3][t] * b[3][l]
                      for l in range(3)] for t in range(3)]
                Ic = (F[0][0] * F[0][0] + F[0][1] * F[0][1] + F[0][2] * F[0][2]
                      + F[1][0] * F[1][0] + F[1][1] * F[1][1] + F[1][2] * F[1][2]
                      + F[2][0] * F[2][0] + F[2][1] * F[2][1] + F[2][2] * F[2][2])
                J = (F[0][0] * (F[1][1] * F[2][2] - F[1][2] * F[2][1])
                     - F[0][1] * (F[1][0] * F[2][2] - F[1][2] * F[2][0])
                     + F[0][2] * (F[1][0] * F[2][1] - F[1][1] * F[2][0]))
                ic_v = jnp.maximum(Ic + 1.0, 0.0) + 1e-30
                alpha = 0.75 * mu / lam + 1.0
                d = J - alpha
                psi = (0.5 * mu * (Ic - 3.0) + 0.5 * lam * (d * d)
                       - 0.5 * mu * _vlog(ic_v))
                acc = acc + jnp.where(valid, psi * meas, 0.0)
            return acc

        acc = lax.fori_loop(0, ITERS, step, acc)

    acc_v[...] = acc
    pltpu.sync_copy(acc_v, out_hbm.at[pl.ds(wid * L, L)])


@jax.jit
def kernel(position, elements, polynomials, lam, mu, measure):
    et = elements.T
    bt = polynomials[:, :, :3].reshape(N_ELEM, NB).T
    px, py, pz = position[:, 0], position[:, 1], position[:, 2]

    mesh = plsc.VectorSubcoreMesh(core_axis_name="c", subcore_axis_name="s",
                                  num_cores=NC, num_subcores=NS)
    partials = pl.kernel(
        _sc_body,
        out_type=jax.ShapeDtypeStruct((NW * L,), jnp.float32),
        mesh=mesh,
        compiler_params=pltpu.CompilerParams(needs_layout_passes=False,
                                             use_tc_tiling_on_sc=False),
        scratch_types=[
            pltpu.VMEM((N_VERT,), jnp.float32),
            pltpu.VMEM((N_VERT,), jnp.float32),
            pltpu.VMEM((N_VERT,), jnp.float32),
            pltpu.VMEM((2, 4, SLOT), jnp.int32),
            pltpu.VMEM((2, NB, SLOT), jnp.float32),
            pltpu.VMEM((2, SLOT), jnp.float32),
            pltpu.VMEM((2, SLOT), jnp.float32),
            pltpu.VMEM((2, SLOT), jnp.float32),
            pltpu.VMEM((L,), jnp.float32),
            pltpu.SemaphoreType.DMA,
            pltpu.SemaphoreType.DMA,
            pltpu.SemaphoreType.DMA,
        ],
    )(et, bt, px, py, pz, lam, mu, measure)
    return jnp.sum(partials)
